# baseline (speedup 1.0000x reference)
import functools

import jax
import jax.numpy as jnp
from jax import lax
from jax.experimental import pallas as pl
from jax.experimental.pallas import tpu as pltpu

K = 32


def _sae_block(x_ref, w_enc_ref, b_enc_ref, w_dec_ref, b_dec_ref, out_ref,
               pre_ref, kv_ref, *, ht, nh, nb, iters):
    i = pl.program_id(0)
    h = pl.program_id(1)
    be = lax.rem(i, 3)
    bs = lax.rem(i + 2, 3)
    bd = lax.rem(i + 1, 3)

    @pl.when(i < nb)
    def _encode():
        xin = x_ref[...] - b_dec_ref[...][None, :]
        pre = lax.dot_general(
            xin, w_enc_ref[...],
            (((1,), (1,)), ((), ())),
            preferred_element_type=jnp.float32,
        )
        pre = jnp.maximum(pre + b_enc_ref[pl.ds(h * ht, ht)][None, :], 0.0)
        pre_ref[be, :, pl.ds(h * ht, ht)] = pre

    @pl.when((i >= 1) & (i <= nb))
    def _select():
        @pl.when(h == 0)
        def _init():
            kv_ref[bs] = jnp.full(kv_ref.shape[1:], jnp.inf, jnp.float32)

        def body(_, m):
            w = pre_ref[bs]
            return jnp.max(jnp.where(w < m, w, -1.0), axis=1, keepdims=True)

        kv_ref[bs] = lax.fori_loop(0, iters, body, kv_ref[bs])

    @pl.when(i >= 2)
    def _decode():
        pre = pre_ref[bd, :, pl.ds(h * ht, ht)]
        sparse = jnp.where(pre >= kv_ref[bd], pre, 0.0)
        acc = lax.dot_general(
            sparse.astype(jnp.bfloat16), w_dec_ref[...],
            (((1,), (0,)), ((), ())),
            preferred_element_type=jnp.float32,
        )

        @pl.when(h == 0)
        def _init():
            out_ref[...] = acc + b_dec_ref[...][None, :]

        @pl.when(h > 0)
        def _accum():
            out_ref[...] = out_ref[...] + acc


@jax.jit
def _sae_forward(x, W_enc, b_enc, W_dec, b_dec):
    n, d_in = x.shape
    hidden = W_enc.shape[0]
    block_rows = 256 if n % 256 == 0 else n
    ht = 768 if hidden % 768 == 0 else hidden
    nb = n // block_rows
    nh = hidden // ht
    iters = -(-K // nh)
    return pl.pallas_call(
        functools.partial(_sae_block, ht=ht, nh=nh, nb=nb, iters=iters),
        grid=(nb + 2, nh),
        in_specs=[
            pl.BlockSpec((block_rows, d_in),
                         lambda i, h: (jnp.minimum(i, nb - 1), 0)),
            pl.BlockSpec((ht, d_in), lambda i, h: (h, 0)),
            pl.BlockSpec((hidden,), lambda i, h: (0,)),
            pl.BlockSpec((ht, d_in), lambda i, h: (h, 0)),
            pl.BlockSpec((d_in,), lambda i, h: (0,)),
        ],
        out_specs=pl.BlockSpec((block_rows, d_in),
                               lambda i, h: (jnp.maximum(i - 2, 0), 0)),
        out_shape=jax.ShapeDtypeStruct((n, d_in), jnp.float32),
        scratch_shapes=[
            pltpu.VMEM((3, block_rows, hidden), jnp.float32),
            pltpu.VMEM((3, block_rows, 1), jnp.float32),
        ],
    )(x, W_enc, b_enc, W_dec.astype(jnp.bfloat16), b_dec)


def kernel(x, W_enc, b_enc, W_dec, b_dec):
    return _sae_forward(x, W_enc, b_enc, W_dec, b_dec)
